# baseline (speedup 1.0000x reference)
import functools
import math

import jax
import jax.numpy as jnp
from jax import lax
from jax.experimental import pallas as pl
from jax.experimental.pallas import tpu as pltpu
from jax.experimental.pallas import tpu_sc as plsc

_VOCAB = 1000000
_D = 64
_BATCH = 16384
_HIST = 200
_BT = _BATCH * _HIST
_NC, _NS = 2, 16
_NW = _NC * _NS
_PER_W = _BT // _NW
_CI = 128
_KI = 8
_CHUNK = _KI * _CI
_N_CHUNK = _PER_W // _CHUNK
_SCALE = math.sqrt(_D)

_mesh = plsc.VectorSubcoreMesh(core_axis_name="c", subcore_axis_name="s")


@functools.partial(
    pl.kernel,
    mesh=_mesh,
    out_type=jax.ShapeDtypeStruct((_BT, _D), jnp.float32),
    scratch_types=[
        pltpu.VMEM((_KI, _CI), jnp.int32),
        pltpu.VMEM((_CHUNK, _D), jnp.float32),
        pltpu.SemaphoreType.DMA,
    ],
    compiler_params=pltpu.CompilerParams(use_tc_tiling_on_sc=False),
)
def _emb_lookup(x_hbm, lut_hbm, out_hbm, idx_v, rows_v, sem):
    wid = lax.axis_index("s") * _NC + lax.axis_index("c")
    base = wid * _PER_W

    def chunk_body(o, carry):
        off = pl.multiple_of(base + o * _CHUNK, _CHUNK)
        row0 = pl.multiple_of(off // _CI, _KI)
        pltpu.sync_copy(x_hbm.at[pl.ds(row0, _KI)], idx_v)
        copies = [
            pltpu.async_copy(
                lut_hbm.at[idx_v.at[j]],
                rows_v.at[pl.ds(j * _CI, _CI)],
                sem,
            )
            for j in range(_KI)
        ]
        for c in copies:
            c.wait()

        def mul_row(r, c2):
            for j4 in range(_D // 16):
                sl = rows_v[r, pl.ds(j4 * 16, 16)]
                rows_v[r, pl.ds(j4 * 16, 16)] = sl * _SCALE
            return c2

        lax.fori_loop(0, _CHUNK, mul_row, 0, unroll=2)
        pltpu.sync_copy(rows_v, out_hbm.at[pl.ds(off, _CHUNK)])
        return carry

    lax.fori_loop(0, _N_CHUNK, chunk_body, 0)


def kernel(x, lut):
    xf = x.reshape(_BT // _CI, _CI).astype(jnp.int32)
    out = _emb_lookup(xf, lut)
    return out.reshape(_BATCH, _HIST, _D)


# speedup vs baseline: 1.1048x; 1.1048x over previous
import functools
import math

import jax
import jax.numpy as jnp
from jax import lax
from jax.experimental import pallas as pl
from jax.experimental.pallas import tpu as pltpu
from jax.experimental.pallas import tpu_sc as plsc

_VOCAB = 1000000
_D = 64
_BATCH = 16384
_HIST = 200
_BT = _BATCH * _HIST
_NC, _NS = 2, 16
_NW = _NC * _NS
_PER_W = _BT // _NW
_CI = 128
_KI = 4
_CHUNK = _KI * _CI
_N_CHUNK = _PER_W // _CHUNK
_NIB = 4
_SCALE = math.sqrt(_D)

_mesh = plsc.VectorSubcoreMesh(core_axis_name="c", subcore_axis_name="s")


@functools.partial(
    pl.kernel,
    mesh=_mesh,
    out_type=jax.ShapeDtypeStruct((_BT, _D), jnp.float32),
    scratch_types=[
        pltpu.VMEM((_NIB, _KI, _CI), jnp.int32),
        pltpu.VMEM((2, _CHUNK, _D), jnp.float32),
        pltpu.SemaphoreType.DMA((_NIB,)),
        pltpu.SemaphoreType.DMA((2,)),
        pltpu.SemaphoreType.DMA((2,)),
    ],
    compiler_params=pltpu.CompilerParams(use_tc_tiling_on_sc=False),
)
def _emb_lookup(x_hbm, lut_hbm, out_hbm, idx_v, rows_v, isem, gsem, ssem):
    wid = lax.axis_index("s") * _NC + lax.axis_index("c")
    cbase = wid * _N_CHUNK
    rbase = wid * _PER_W

    def fire_gathers(g, buf):
        for j in range(_KI):
            pltpu.async_copy(
                lut_hbm.at[idx_v.at[g % _NIB, j]],
                rows_v.at[buf, pl.ds(j * _CI, _CI)],
                gsem.at[buf],
            )

    def drain_gathers(buf):
        for j in range(_KI):
            pltpu.make_async_copy(
                lut_hbm.at[pl.ds(0, _CI)],
                rows_v.at[buf, pl.ds(j * _CI, _CI)],
                gsem.at[buf],
            ).wait()

    def drain_store(buf):
        pltpu.make_async_copy(
            rows_v.at[buf],
            out_hbm.at[pl.ds(0, _CHUNK)],
            ssem.at[buf],
        ).wait()

    def drain_idx(slot):
        pltpu.make_async_copy(
            x_hbm.at[0],
            idx_v.at[slot],
            isem.at[slot],
        ).wait()

    pltpu.sync_copy(x_hbm.at[cbase], idx_v.at[0])
    pltpu.sync_copy(x_hbm.at[cbase + 1], idx_v.at[1])
    pltpu.sync_copy(x_hbm.at[cbase + 2], idx_v.at[2])
    fire_gathers(0, 0)
    fire_gathers(1, 1)

    def chunk_body(g, carry):
        a = lax.rem(g, 2)
        b = lax.rem(g + 1, 2)

        @pl.when(g + 3 < _N_CHUNK)
        def _():
            slot = lax.rem(g + 3, _NIB)
            pltpu.async_copy(x_hbm.at[cbase + g + 3], idx_v.at[slot],
                             isem.at[slot])

        @pl.when(jnp.logical_and(g >= 1, g + 1 < _N_CHUNK))
        def _():
            drain_store(b)

            @pl.when(g + 1 >= 3)
            def _():
                drain_idx(lax.rem(g + 1, _NIB))

            fire_gathers(g + 1, b)

        drain_gathers(a)

        def mul_row(r, c2):
            for j4 in range(_D // 16):
                sl = rows_v[a, r, pl.ds(j4 * 16, 16)]
                rows_v[a, r, pl.ds(j4 * 16, 16)] = sl * _SCALE
            return c2

        lax.fori_loop(0, _CHUNK, mul_row, 0, unroll=4)
        off = pl.multiple_of(rbase + g * _CHUNK, _CHUNK)
        pltpu.async_copy(rows_v.at[a], out_hbm.at[pl.ds(off, _CHUNK)],
                         ssem.at[a])
        return carry

    lax.fori_loop(0, _N_CHUNK, chunk_body, 0)
    drain_store(lax.rem(_N_CHUNK - 1, 2))
    drain_store(lax.rem(_N_CHUNK, 2))


def kernel(x, lut):
    xf = x.reshape(_BT // _CHUNK, _KI, _CI).astype(jnp.int32)
    out = _emb_lookup(xf, lut)
    return out.reshape(_BATCH, _HIST, _D)


# speedup vs baseline: 1.1058x; 1.0009x over previous
import functools
import math

import jax
import jax.numpy as jnp
from jax import lax
from jax.experimental import pallas as pl
from jax.experimental.pallas import tpu as pltpu
from jax.experimental.pallas import tpu_sc as plsc

_VOCAB = 1000000
_D = 64
_BATCH = 16384
_HIST = 200
_BT = _BATCH * _HIST
_NC, _NS = 2, 16
_NW = _NC * _NS
_PER_W = _BT // _NW
_CI = 128
_KI = 4
_CHUNK = _KI * _CI
_N_CHUNK = _PER_W // _CHUNK
_NIB = 4
_SCALE = math.sqrt(_D)

_mesh = plsc.VectorSubcoreMesh(core_axis_name="c", subcore_axis_name="s")


@functools.partial(
    pl.kernel,
    mesh=_mesh,
    out_type=jax.ShapeDtypeStruct((_BT, _D), jnp.float32),
    scratch_types=[
        pltpu.VMEM((_NIB, _KI, _CI), jnp.int32),
        pltpu.VMEM((2, _CHUNK, _D), jnp.float32),
        pltpu.SemaphoreType.DMA((_NIB,)),
        pltpu.SemaphoreType.DMA((2,)),
        pltpu.SemaphoreType.DMA((2,)),
    ],
    compiler_params=pltpu.CompilerParams(use_tc_tiling_on_sc=False),
)
def _emb_lookup(x_hbm, lut_hbm, out_hbm, idx_v, rows_v, isem, gsem, ssem):
    wid = lax.axis_index("s") * _NC + lax.axis_index("c")
    cbase = wid * _N_CHUNK
    rbase = wid * _PER_W

    def fire_gathers(g, buf):
        for j in range(_KI):
            pltpu.async_copy(
                lut_hbm.at[idx_v.at[g % _NIB, j]],
                rows_v.at[buf, pl.ds(j * _CI, _CI)],
                gsem.at[buf],
            )

    def drain_gathers(buf):
        for j in range(_KI):
            pltpu.make_async_copy(
                lut_hbm.at[pl.ds(0, _CI)],
                rows_v.at[buf, pl.ds(j * _CI, _CI)],
                gsem.at[buf],
            ).wait()

    def drain_store(buf):
        pltpu.make_async_copy(
            rows_v.at[buf],
            out_hbm.at[pl.ds(0, _CHUNK)],
            ssem.at[buf],
        ).wait()

    def drain_idx(slot):
        pltpu.make_async_copy(
            x_hbm.at[0],
            idx_v.at[slot],
            isem.at[slot],
        ).wait()

    pltpu.sync_copy(x_hbm.at[cbase], idx_v.at[0])
    pltpu.sync_copy(x_hbm.at[cbase + 1], idx_v.at[1])
    pltpu.sync_copy(x_hbm.at[cbase + 2], idx_v.at[2])
    fire_gathers(0, 0)
    fire_gathers(1, 1)

    def chunk_body(g, carry):
        a = lax.rem(g, 2)
        b = lax.rem(g + 1, 2)

        @pl.when(g + 3 < _N_CHUNK)
        def _():
            slot = lax.rem(g + 3, _NIB)
            pltpu.async_copy(x_hbm.at[cbase + g + 3], idx_v.at[slot],
                             isem.at[slot])

        @pl.when(jnp.logical_and(g >= 1, g + 1 < _N_CHUNK))
        def _():
            drain_store(b)

            @pl.when(g + 1 >= 3)
            def _():
                drain_idx(lax.rem(g + 1, _NIB))

            fire_gathers(g + 1, b)

        drain_gathers(a)

        @plsc.parallel_loop(0, _CHUNK, step=1, unroll=8)
        def _(r):
            for j4 in range(_D // 16):
                sl = rows_v[a, r, pl.ds(j4 * 16, 16)]
                rows_v[a, r, pl.ds(j4 * 16, 16)] = sl * _SCALE
        off = pl.multiple_of(rbase + g * _CHUNK, _CHUNK)
        pltpu.async_copy(rows_v.at[a], out_hbm.at[pl.ds(off, _CHUNK)],
                         ssem.at[a])
        return carry

    lax.fori_loop(0, _N_CHUNK, chunk_body, 0)
    drain_store(lax.rem(_N_CHUNK - 1, 2))
    drain_store(lax.rem(_N_CHUNK, 2))


def kernel(x, lut):
    xf = x.reshape(_BT // _CHUNK, _KI, _CI).astype(jnp.int32)
    out = _emb_lookup(xf, lut)
    return out.reshape(_BATCH, _HIST, _D)
